# Initial kernel scaffold; baseline (speedup 1.0000x reference)
import jax
import jax.numpy as jnp
from jax.experimental import pallas as pl


def kernel(x, W_g1, b_g1, W_g2, b_g2, W_a1, b_a1, W_a2, b_a2, W_b1, b_b1, W_b2, b_b2):
    raise NotImplementedError("write your pallas kernel here")


# speedup vs baseline: 1.6963x; 1.6963x over previous
import functools

import jax
import jax.numpy as jnp
from jax.experimental import pallas as pl


def _moe_body(xt_ref, xb_ref, w1t_ref, w1b_ref, b1_ref, w2_ref, b2_ref, out_ref):
    xt = xt_ref[...].astype(jnp.bfloat16)
    xb = xb_ref[...].astype(jnp.bfloat16)
    h = jnp.dot(xt, w1t_ref[...], preferred_element_type=jnp.float32)
    h = h + jnp.dot(xb, w1b_ref[...], preferred_element_type=jnp.float32)
    h = jnp.maximum(h + b1_ref[...], 0.0).astype(jnp.bfloat16)
    y = jnp.dot(h, w2_ref[...],
                preferred_element_type=jnp.float32) + b2_ref[...]
    ga = y[:, 4:5]
    gb = y[:, 5:6]
    ma = jnp.clip(jnp.sign(ga - gb) + 1.0, 0.0, 1.0)
    out_ref[:, 0:2] = y[:, 0:2] * ma
    out_ref[:, 2:4] = y[:, 2:4] * (1.0 - ma)


def kernel(x, W_g1, b_g1, W_g2, b_g2, W_a1, b_a1, W_a2, b_a2, W_b1, b_b1, W_b2, b_b2):
    n, d = x.shape
    m = n // 2
    bm = 2048
    grid = m // bm

    W1 = jnp.concatenate([W_g1, W_a1, W_b1], axis=1)
    W1t = W1[:d].astype(jnp.bfloat16)
    W1b = W1[d:].astype(jnp.bfloat16)
    b1 = jnp.concatenate([b_g1, b_a1, b_b1]).reshape(1, -1)

    W2 = jnp.zeros((320, 6), dtype=x.dtype)
    W2 = W2.at[0:64, 4:6].set(W_g2)
    W2 = W2.at[64:192, 0:2].set(W_a2)
    W2 = W2.at[192:320, 2:4].set(W_b2)
    W2 = W2.astype(jnp.bfloat16)
    b2 = jnp.concatenate([b_a2, b_b2, b_g2]).reshape(1, -1)

    nblk = grid
    return pl.pallas_call(
        _moe_body,
        grid=(grid,),
        in_specs=[
            pl.BlockSpec((bm, d), lambda i: (i, 0)),
            pl.BlockSpec((bm, d), lambda i, _n=nblk: (i + _n, 0)),
            pl.BlockSpec((d, 320), lambda i: (0, 0)),
            pl.BlockSpec((d, 320), lambda i: (0, 0)),
            pl.BlockSpec((1, 320), lambda i: (0, 0)),
            pl.BlockSpec((320, 6), lambda i: (0, 0)),
            pl.BlockSpec((1, 6), lambda i: (0, 0)),
        ],
        out_specs=pl.BlockSpec((bm, 4), lambda i: (i, 0)),
        out_shape=jax.ShapeDtypeStruct((m, 4), x.dtype),
    )(x, x, W1t, W1b, b1, W2, b2)
